# baseline (speedup 1.0000x reference)
import functools

import jax
import jax.numpy as jnp
from jax import lax
from jax.experimental import pallas as pl
from jax.experimental.pallas import tpu as pltpu
from jax.experimental.pallas import tpu_sc as plsc

GAT_HEADS = 4
T = 25
BETA = 0.9
HID = 128
D_HEAD = HID // GAT_HEADS
NEG_SLOPE = 0.2
THRESH = 1.0

N = 10000
NPAD = 10112
G = 128
NSUB = G // 16
NWORKERS = 32
GP = 84
E_PAD = NWORKERS * GP * G
E_ALLOC = E_PAD + 2 * G
ROWS_PER_SUB = NPAD // 16
DEN_W = 8
ATAB_ROWS = N + 16


def _mm_body(x_ref, w_ref, asrc_w_ref, adst_w_ref, h_ref, atab_ref):
    h = jnp.dot(x_ref[...], w_ref[...], preferred_element_type=jnp.float32)
    h_ref[...] = h
    a_src = jnp.dot(h, asrc_w_ref[...], preferred_element_type=jnp.float32)
    a_dst = jnp.dot(h, adst_w_ref[...], preferred_element_type=jnp.float32)
    atab_ref[...] = jnp.concatenate([a_src, a_dst], axis=1)


def _matmul_attn(x, W, att_src, att_dst):
    N_, D_IN = x.shape
    BN = 2000
    asrc_w = jnp.zeros((HID, GAT_HEADS), jnp.float32)
    adst_w = jnp.zeros((HID, GAT_HEADS), jnp.float32)
    for h in range(GAT_HEADS):
        asrc_w = asrc_w.at[h * D_HEAD:(h + 1) * D_HEAD, h].set(att_src[0, h])
        adst_w = adst_w.at[h * D_HEAD:(h + 1) * D_HEAD, h].set(att_dst[0, h])
    return pl.pallas_call(
        _mm_body,
        grid=(N_ // BN,),
        in_specs=[
            pl.BlockSpec((BN, D_IN), lambda i: (i, 0)),
            pl.BlockSpec((D_IN, HID), lambda i: (0, 0)),
            pl.BlockSpec((HID, GAT_HEADS), lambda i: (0, 0)),
            pl.BlockSpec((HID, GAT_HEADS), lambda i: (0, 0)),
        ],
        out_specs=[
            pl.BlockSpec((BN, HID), lambda i: (i, 0)),
            pl.BlockSpec((BN, 2 * GAT_HEADS), lambda i: (i, 0)),
        ],
        out_shape=[
            jax.ShapeDtypeStruct((N_, HID), jnp.float32),
            jax.ShapeDtypeStruct((N_, 2 * GAT_HEADS), jnp.float32),
        ],
    )(x, W, asrc_w, adst_w)


def _edge_body(src_hbm, dst_hbm, h_hbm, atab_hbm, num_hbm, den_hbm,
               srcb0, srcb1, srcb2, srcb3, dstb0, dstb1, dstb2, dstb3, jidx,
               hbuf0, hbuf1, asrcb0, asrcb1, adstb0, adstb1,
               exbuf0, exbuf1, exflat0, exflat1, acc_sh, den_sh,
               isem0, isem1, isem2, isem3, gsem0, gsem1, asem0, asem1,
               ssem0, ssem1):
    srcb = (srcb0, srcb1, srcb2, srcb3)
    dstb = (dstb0, dstb1, dstb2, dstb3)
    hbuf = (hbuf0, hbuf1)
    asrcb = (asrcb0, asrcb1)
    adstb = (adstb0, adstb1)
    exbuf = (exbuf0, exbuf1)
    exflat = (exflat0, exflat1)
    isem = (isem0, isem1, isem2, isem3)
    gsem = (gsem0, gsem1)
    asem = (asem0, asem1)
    ssem = (ssem0, ssem1)

    c = lax.axis_index("c")
    s = lax.axis_index("s")
    wid = s * 2 + c
    zero16 = jnp.zeros((16,), jnp.float32)
    iota16 = lax.iota(jnp.int32, 16)
    ebase = wid * GP * G

    for b in range(2):
        def _zrow(j, _, _b=b):
            for gi in range(8):
                hbuf[_b][j, pl.ds(gi * 16, 16)] = zero16
            return 0
        lax.fori_loop(0, G, _zrow, 0)
        for sub in range(NSUB):
            for col in range(DEN_W):
                plsc.store_scatter(
                    exbuf[b],
                    [sub * 16 + iota16, jnp.full((16,), col, jnp.int32)],
                    zero16)
    for sub in range(NSUB):
        jidx[pl.ds(sub * 16, 16)] = jnp.full((16,), N, jnp.int32)
    base = s * ROWS_PER_SUB
    off = 0
    while off < ROWS_PER_SUB:
        sz = min(G, ROWS_PER_SUB - off)
        pltpu.sync_copy(hbuf0.at[pl.ds(0, sz)], acc_sh.at[pl.ds(base + off, sz)])
        pltpu.sync_copy(exbuf0.at[pl.ds(0, sz)], den_sh.at[pl.ds(base + off, sz)])
        off += sz
    plsc.subcore_barrier()

    def stage_a(g, b4):
        eoff = ebase + g * G
        pltpu.async_copy(src_hbm.at[pl.ds(eoff, G)], srcb[b4], isem[b4])
        pltpu.async_copy(dst_hbm.at[pl.ds(eoff, G)], dstb[b4], isem[b4])

    def drain_scatter(b2):
        pltpu.make_async_copy(h_hbm.at[pl.ds(0, G)], hbuf[b2], ssem[b2]).wait()
        pltpu.make_async_copy(den_hbm.at[0, pl.ds(0, G)], exbuf[b2],
                              ssem[b2]).wait()

    def stage_b(b4, b2):
        drain_scatter(b2)
        pltpu.make_async_copy(src_hbm.at[pl.ds(0, G)], srcb[b4], isem[b4]).wait()
        pltpu.make_async_copy(dst_hbm.at[pl.ds(0, G)], dstb[b4], isem[b4]).wait()
        pltpu.async_copy(h_hbm.at[srcb[b4]], hbuf[b2], gsem[b2])
        pltpu.async_copy(atab_hbm.at[srcb[b4]], asrcb[b2], asem[b2])
        pltpu.async_copy(atab_hbm.at[dstb[b4]], adstb[b2], asem[b2])

    def stage_c(b4, b2):
        pltpu.make_async_copy(atab_hbm.at[pl.ds(0, G)], asrcb[b2], asem[b2]).wait()
        pltpu.make_async_copy(atab_hbm.at[pl.ds(0, G)], adstb[b2], asem[b2]).wait()
        for sub in range(NSUB):
            evec = sub * 16 + iota16
            for head in range(GAT_HEADS):
                hvec = jnp.full((16,), head, jnp.int32)
                asrc = plsc.load_gather(asrcb[b2], [evec, hvec])
                adst = plsc.load_gather(adstb[b2], [evec, hvec + GAT_HEADS])
                e = asrc + adst
                e = jnp.where(e > 0, e, NEG_SLOPE * e)
                ex = jnp.exp(e)
                plsc.store_scatter(exbuf[b2], [evec, hvec], ex)
                plsc.store_scatter(exflat[b2], [evec * 4 + head], ex)
        pltpu.make_async_copy(h_hbm.at[pl.ds(0, G)], hbuf[b2], gsem[b2]).wait()

        def _squad(q, _):
            exv = exflat[b2][pl.ds(q * 16, 16)]
            for k in range(4):
                j = q * 4 + k
                for gi in range(8):
                    w = jnp.broadcast_to(exv[k * 4 + gi // 2], (16,))
                    hbuf[b2][j, pl.ds(gi * 16, 16)] = (
                        hbuf[b2][j, pl.ds(gi * 16, 16)] * w)
            return 0
        lax.fori_loop(0, G // 4, _squad, 0, unroll=4)
        pltpu.async_copy(hbuf[b2], acc_sh.at[dstb[b4]], ssem[b2], add=True)
        pltpu.async_copy(exbuf[b2], den_sh.at[dstb[b4]], ssem[b2], add=True)

    pltpu.async_copy(hbuf0, acc_sh.at[jidx], ssem0, add=True)
    pltpu.async_copy(exbuf0, den_sh.at[jidx], ssem0, add=True)
    pltpu.async_copy(hbuf1, acc_sh.at[jidx], ssem1, add=True)
    pltpu.async_copy(exbuf1, den_sh.at[jidx], ssem1, add=True)
    stage_a(0, 0)
    stage_a(1, 1)
    stage_b(0, 0)

    def _quad(i, _):
        g = i * 4
        for k in range(4):
            stage_a(g + k + 2, (k + 2) % 4)
            stage_b((k + 1) % 4, (k + 1) % 2)
            stage_c(k % 4, k % 2)
        return 0
    lax.fori_loop(0, GP // 4, _quad, 0)

    drain_scatter(1)
    pltpu.make_async_copy(src_hbm.at[pl.ds(0, G)], srcb1, isem1).wait()
    pltpu.make_async_copy(dst_hbm.at[pl.ds(0, G)], dstb1, isem1).wait()
    pltpu.make_async_copy(h_hbm.at[pl.ds(0, G)], hbuf0, gsem0).wait()
    pltpu.make_async_copy(atab_hbm.at[pl.ds(0, G)], asrcb0, asem0).wait()
    pltpu.make_async_copy(atab_hbm.at[pl.ds(0, G)], adstb0, asem0).wait()
    plsc.subcore_barrier()
    pltpu.sync_copy(acc_sh.at[pl.ds(base, ROWS_PER_SUB)],
                    num_hbm.at[c, pl.ds(base, ROWS_PER_SUB)])
    pltpu.sync_copy(den_sh.at[pl.ds(base, ROWS_PER_SUB)],
                    den_hbm.at[c, pl.ds(base, ROWS_PER_SUB)])


def _edge_pass(src, dst, h, atab):
    mesh = plsc.VectorSubcoreMesh(core_axis_name="c", subcore_axis_name="s")
    f = pl.kernel(
        _edge_body,
        out_type=[
            jax.ShapeDtypeStruct((2, NPAD, HID), jnp.float32),
            jax.ShapeDtypeStruct((2, NPAD, DEN_W), jnp.float32),
        ],
        mesh=mesh,
        compiler_params=pltpu.CompilerParams(
            needs_layout_passes=False, use_tc_tiling_on_sc=False),
        scratch_types=(
            [pltpu.VMEM((G,), jnp.int32) for _ in range(9)]
            + [pltpu.VMEM((G, HID), jnp.float32) for _ in range(2)]
            + [pltpu.VMEM((G, 2 * GAT_HEADS), jnp.float32) for _ in range(4)]
            + [pltpu.VMEM((G, DEN_W), jnp.float32) for _ in range(2)]
            + [pltpu.VMEM((G * GAT_HEADS,), jnp.float32) for _ in range(2)]
            + [pltpu.VMEM_SHARED((NPAD, HID), jnp.float32),
               pltpu.VMEM_SHARED((NPAD, DEN_W), jnp.float32)]
            + [pltpu.SemaphoreType.DMA for _ in range(10)]
        ),
    )
    return f(src, dst, h, atab)


def _spike_body(num_ref, den_ref, x_ref, bias_ref, out_ref):
    den = (den_ref[0] + den_ref[1])[:, :GAT_HEADS]
    g = (num_ref[0] + num_ref[1]) / jnp.repeat(den + 1e-16, D_HEAD, axis=1)
    g = g + bias_ref[...]
    mem = jnp.zeros_like(g)
    spk_accum = jnp.zeros_like(g)
    for _ in range(T):
        reset = (mem > THRESH).astype(jnp.float32)
        mem = BETA * mem + g - reset * THRESH
        spk_accum = spk_accum + (mem > THRESH).astype(jnp.float32)
    out_ref[...] = spk_accum * (1.0 / T) + x_ref[...]


def _spike_phase(num2, den2, x, bias):
    BN = 2000
    return pl.pallas_call(
        _spike_body,
        grid=(N // BN,),
        in_specs=[
            pl.BlockSpec((2, BN, HID), lambda i: (0, i, 0)),
            pl.BlockSpec((2, BN, DEN_W), lambda i: (0, i, 0)),
            pl.BlockSpec((BN, HID), lambda i: (i, 0)),
            pl.BlockSpec((1, HID), lambda i: (0, 0)),
        ],
        out_specs=pl.BlockSpec((BN, HID), lambda i: (i, 0)),
        out_shape=jax.ShapeDtypeStruct((N, HID), jnp.float32),
    )(num2, den2, x, bias.reshape(1, HID))


def kernel(x, edge_index, W, att_src, att_dst, bias):
    h, atab = _matmul_attn(x, W, att_src, att_dst)
    atab_ext = jnp.concatenate(
        [atab, jnp.full((ATAB_ROWS - N, 2 * GAT_HEADS), -1e30, jnp.float32)])
    loop = jnp.arange(N, dtype=edge_index.dtype)
    npad_e = E_ALLOC - edge_index.shape[1] - N
    src = jnp.concatenate(
        [edge_index[0], loop, jnp.zeros((npad_e,), edge_index.dtype)])
    dst = jnp.concatenate(
        [edge_index[1], loop, jnp.full((npad_e,), N, edge_index.dtype)])
    num2, den2 = _edge_pass(src, dst, h, atab_ext)
    return _spike_phase(num2, den2, x, bias)


# speedup vs baseline: 2.1421x; 2.1421x over previous
import functools

import jax
import jax.numpy as jnp
from jax import lax
from jax.experimental import pallas as pl
from jax.experimental.pallas import tpu as pltpu
from jax.experimental.pallas import tpu_sc as plsc

GAT_HEADS = 4
T = 25
BETA = 0.9
HID = 128
D_HEAD = HID // GAT_HEADS
NEG_SLOPE = 0.2
THRESH = 1.0

N = 10000
NPAD = 10112
G = 96
NSUB = G // 16
NWORKERS = 32
GP = 108
E_PAD = NWORKERS * GP * G
E_ALLOC = E_PAD + 2 * G
ROWS_PER_SUB = NPAD // 16
DEN_W = 8
ATAB_ROWS = N + 16


def _mm_body(x_ref, w_ref, asrc_w_ref, adst_w_ref, h_ref, atab_ref):
    h = jnp.dot(x_ref[...], w_ref[...], preferred_element_type=jnp.float32)
    h_ref[...] = h
    a_src = jnp.dot(h, asrc_w_ref[...], preferred_element_type=jnp.float32)
    a_dst = jnp.dot(h, adst_w_ref[...], preferred_element_type=jnp.float32)
    atab_ref[...] = jnp.concatenate([a_src, a_dst], axis=1)


def _matmul_attn(x, W, att_src, att_dst):
    N_, D_IN = x.shape
    BN = 2000
    asrc_w = jnp.zeros((HID, GAT_HEADS), jnp.float32)
    adst_w = jnp.zeros((HID, GAT_HEADS), jnp.float32)
    for h in range(GAT_HEADS):
        asrc_w = asrc_w.at[h * D_HEAD:(h + 1) * D_HEAD, h].set(att_src[0, h])
        adst_w = adst_w.at[h * D_HEAD:(h + 1) * D_HEAD, h].set(att_dst[0, h])
    return pl.pallas_call(
        _mm_body,
        grid=(N_ // BN,),
        in_specs=[
            pl.BlockSpec((BN, D_IN), lambda i: (i, 0)),
            pl.BlockSpec((D_IN, HID), lambda i: (0, 0)),
            pl.BlockSpec((HID, GAT_HEADS), lambda i: (0, 0)),
            pl.BlockSpec((HID, GAT_HEADS), lambda i: (0, 0)),
        ],
        out_specs=[
            pl.BlockSpec((BN, HID), lambda i: (i, 0)),
            pl.BlockSpec((BN, 2 * GAT_HEADS), lambda i: (i, 0)),
        ],
        out_shape=[
            jax.ShapeDtypeStruct((N_, HID), jnp.float32),
            jax.ShapeDtypeStruct((N_, 2 * GAT_HEADS), jnp.float32),
        ],
    )(x, W, asrc_w, adst_w)


def _edge_body(src_hbm, dst_hbm, h_hbm, atab_hbm, num_hbm, den_hbm,
               srcb0, srcb1, srcb2, srcb3, dstb0, dstb1, dstb2, dstb3, jidx,
               hbuf0, hbuf1, asrcb0, asrcb1, adstb0, adstb1,
               exbuf0, exbuf1, exflat0, exflat1, acc_sh, den_sh,
               isem0, isem1, isem2, isem3, gsem0, gsem1, asem0, asem1,
               ssem0, ssem1):
    srcb = (srcb0, srcb1, srcb2, srcb3)
    dstb = (dstb0, dstb1, dstb2, dstb3)
    hbuf = (hbuf0, hbuf1)
    asrcb = (asrcb0, asrcb1)
    adstb = (adstb0, adstb1)
    exbuf = (exbuf0, exbuf1)
    exflat = (exflat0, exflat1)
    isem = (isem0, isem1, isem2, isem3)
    gsem = (gsem0, gsem1)
    asem = (asem0, asem1)
    ssem = (ssem0, ssem1)

    c = lax.axis_index("c")
    s = lax.axis_index("s")
    wid = s * 2 + c
    zero16 = jnp.zeros((16,), jnp.float32)
    iota16 = lax.iota(jnp.int32, 16)
    ebase = wid * GP * G

    for b in range(2):
        def _zrow(j, _, _b=b):
            for gi in range(8):
                hbuf[_b][j, pl.ds(gi * 16, 16)] = zero16
            return 0
        lax.fori_loop(0, G, _zrow, 0)
        for sub in range(NSUB):
            for col in range(DEN_W):
                plsc.store_scatter(
                    exbuf[b],
                    [sub * 16 + iota16, jnp.full((16,), col, jnp.int32)],
                    zero16)
    for sub in range(NSUB):
        jidx[pl.ds(sub * 16, 16)] = jnp.full((16,), N, jnp.int32)
    base = s * ROWS_PER_SUB
    off = 0
    while off < ROWS_PER_SUB:
        sz = min(G, ROWS_PER_SUB - off)
        pltpu.sync_copy(hbuf0.at[pl.ds(0, sz)], acc_sh.at[pl.ds(base + off, sz)])
        pltpu.sync_copy(exbuf0.at[pl.ds(0, sz)], den_sh.at[pl.ds(base + off, sz)])
        off += sz
    plsc.subcore_barrier()

    def stage_a(g, b4):
        eoff = ebase + g * G
        pltpu.async_copy(src_hbm.at[pl.ds(eoff, G)], srcb[b4], isem[b4])
        pltpu.async_copy(dst_hbm.at[pl.ds(eoff, G)], dstb[b4], isem[b4])

    def drain_scatter(b2):
        pltpu.make_async_copy(h_hbm.at[pl.ds(0, G)], hbuf[b2], ssem[b2]).wait()
        pltpu.make_async_copy(den_hbm.at[0, pl.ds(0, G)], exbuf[b2],
                              ssem[b2]).wait()

    def stage_b(b4, b2):
        drain_scatter(b2)
        pltpu.make_async_copy(src_hbm.at[pl.ds(0, G)], srcb[b4], isem[b4]).wait()
        pltpu.make_async_copy(dst_hbm.at[pl.ds(0, G)], dstb[b4], isem[b4]).wait()
        pltpu.async_copy(h_hbm.at[srcb[b4]], hbuf[b2], gsem[b2])
        pltpu.async_copy(atab_hbm.at[srcb[b4]], asrcb[b2], asem[b2])
        pltpu.async_copy(atab_hbm.at[dstb[b4]], adstb[b2], asem[b2])

    def stage_c(b4, b2):
        pltpu.make_async_copy(atab_hbm.at[pl.ds(0, G)], asrcb[b2], asem[b2]).wait()
        pltpu.make_async_copy(atab_hbm.at[pl.ds(0, G)], adstb[b2], asem[b2]).wait()
        for sub in range(NSUB):
            evec = sub * 16 + iota16
            for head in range(GAT_HEADS):
                hvec = jnp.full((16,), head, jnp.int32)
                asrc = plsc.load_gather(asrcb[b2], [evec, hvec])
                adst = plsc.load_gather(adstb[b2], [evec, hvec + GAT_HEADS])
                e = asrc + adst
                e = jnp.where(e > 0, e, NEG_SLOPE * e)
                ex = jnp.exp(e)
                plsc.store_scatter(exbuf[b2], [evec, hvec], ex)
                plsc.store_scatter(exflat[b2], [evec * 4 + head], ex)
        pltpu.make_async_copy(h_hbm.at[pl.ds(0, G)], hbuf[b2], gsem[b2]).wait()

        def _squad(q, _):
            exv = exflat[b2][pl.ds(q * 16, 16)]
            for k in range(4):
                j = q * 4 + k
                for gi in range(8):
                    w = jnp.broadcast_to(exv[k * 4 + gi // 2], (16,))
                    hbuf[b2][j, pl.ds(gi * 16, 16)] = (
                        hbuf[b2][j, pl.ds(gi * 16, 16)] * w)
            return 0
        lax.fori_loop(0, G // 4, _squad, 0, unroll=4)
        pltpu.async_copy(hbuf[b2], acc_sh.at[dstb[b4]], ssem[b2], add=True)
        pltpu.async_copy(exbuf[b2], den_sh.at[dstb[b4]], ssem[b2], add=True)

    pltpu.async_copy(hbuf0, acc_sh.at[jidx], ssem0, add=True)
    pltpu.async_copy(exbuf0, den_sh.at[jidx], ssem0, add=True)
    pltpu.async_copy(hbuf1, acc_sh.at[jidx], ssem1, add=True)
    pltpu.async_copy(exbuf1, den_sh.at[jidx], ssem1, add=True)
    stage_a(0, 0)
    stage_a(1, 1)
    stage_b(0, 0)

    def _quad(i, _):
        g = i * 4
        for k in range(4):
            stage_a(g + k + 2, (k + 2) % 4)
            stage_b((k + 1) % 4, (k + 1) % 2)
            stage_c(k % 4, k % 2)
        return 0
    lax.fori_loop(0, GP // 4, _quad, 0)

    drain_scatter(1)
    pltpu.make_async_copy(src_hbm.at[pl.ds(0, G)], srcb1, isem1).wait()
    pltpu.make_async_copy(dst_hbm.at[pl.ds(0, G)], dstb1, isem1).wait()
    pltpu.make_async_copy(h_hbm.at[pl.ds(0, G)], hbuf0, gsem0).wait()
    pltpu.make_async_copy(atab_hbm.at[pl.ds(0, G)], asrcb0, asem0).wait()
    pltpu.make_async_copy(atab_hbm.at[pl.ds(0, G)], adstb0, asem0).wait()
    plsc.subcore_barrier()
    pltpu.sync_copy(acc_sh.at[pl.ds(base, ROWS_PER_SUB)],
                    num_hbm.at[c, pl.ds(base, ROWS_PER_SUB)])
    pltpu.sync_copy(den_sh.at[pl.ds(base, ROWS_PER_SUB)],
                    den_hbm.at[c, pl.ds(base, ROWS_PER_SUB)])


def _edge_pass(src, dst, h, atab):
    mesh = plsc.VectorSubcoreMesh(core_axis_name="c", subcore_axis_name="s")
    f = pl.kernel(
        _edge_body,
        out_type=[
            jax.ShapeDtypeStruct((2, NPAD, HID), jnp.float32),
            jax.ShapeDtypeStruct((2, NPAD, DEN_W), jnp.float32),
        ],
        mesh=mesh,
        compiler_params=pltpu.CompilerParams(
            needs_layout_passes=False, use_tc_tiling_on_sc=False),
        scratch_types=(
            [pltpu.VMEM((G,), jnp.int32) for _ in range(9)]
            + [pltpu.VMEM((G, HID), jnp.float32) for _ in range(2)]
            + [pltpu.VMEM((G, 2 * GAT_HEADS), jnp.float32) for _ in range(4)]
            + [pltpu.VMEM((G, DEN_W), jnp.float32) for _ in range(2)]
            + [pltpu.VMEM((G * GAT_HEADS,), jnp.float32) for _ in range(2)]
            + [pltpu.VMEM_SHARED((NPAD, HID), jnp.float32),
               pltpu.VMEM_SHARED((NPAD, DEN_W), jnp.float32)]
            + [pltpu.SemaphoreType.DMA for _ in range(10)]
        ),
    )
    return f(src, dst, h, atab)


def _spike_body(num_ref, den_ref, x_ref, bias_ref, out_ref):
    den = (den_ref[0] + den_ref[1])[:, :GAT_HEADS]
    g = (num_ref[0] + num_ref[1]) / jnp.repeat(den + 1e-16, D_HEAD, axis=1)
    g = g + bias_ref[...]
    mem = jnp.zeros_like(g)
    spk_accum = jnp.zeros_like(g)
    for _ in range(T):
        reset = (mem > THRESH).astype(jnp.float32)
        mem = BETA * mem + g - reset * THRESH
        spk_accum = spk_accum + (mem > THRESH).astype(jnp.float32)
    out_ref[...] = spk_accum * (1.0 / T) + x_ref[...]


def _spike_phase(num2, den2, x, bias):
    BN = 2000
    return pl.pallas_call(
        _spike_body,
        grid=(N // BN,),
        in_specs=[
            pl.BlockSpec((2, BN, HID), lambda i: (0, i, 0)),
            pl.BlockSpec((2, BN, DEN_W), lambda i: (0, i, 0)),
            pl.BlockSpec((BN, HID), lambda i: (i, 0)),
            pl.BlockSpec((1, HID), lambda i: (0, 0)),
        ],
        out_specs=pl.BlockSpec((BN, HID), lambda i: (i, 0)),
        out_shape=jax.ShapeDtypeStruct((N, HID), jnp.float32),
    )(num2, den2, x, bias.reshape(1, HID))


def kernel(x, edge_index, W, att_src, att_dst, bias):
    h, atab = _matmul_attn(x, W, att_src, att_dst)
    atab_ext = jnp.concatenate(
        [atab, jnp.full((ATAB_ROWS - N, 2 * GAT_HEADS), -1e30, jnp.float32)])
    loop = jnp.arange(N, dtype=edge_index.dtype)
    npad_e = E_ALLOC - edge_index.shape[1] - N
    src = jnp.concatenate(
        [edge_index[0], loop, jnp.zeros((npad_e,), edge_index.dtype)])
    dst = jnp.concatenate(
        [edge_index[1], loop, jnp.full((npad_e,), N, edge_index.dtype)])
    num2, den2 = _edge_pass(src, dst, h, atab_ext)
    return _spike_phase(num2, den2, x, bias)
